# Initial kernel scaffold; baseline (speedup 1.0000x reference)
import jax
import jax.numpy as jnp
from jax.experimental import pallas as pl


def kernel(x, edge_index, Wl0, bl0, Wr0, Wl1, bl1, Wr1, Wl2, bl2, Wr2, g0, b0, g1, b1):
    raise NotImplementedError("write your pallas kernel here")


# speedup vs baseline: 4.5804x; 4.5804x over previous
import jax
import jax.numpy as jnp
from jax import lax
from jax.experimental import pallas as pl
from jax.experimental.pallas import tpu as pltpu
from jax.experimental.pallas import tpu_sc as plsc

_N = 10000
_E = 320000
_D = 128
_NC = 2
_NS = 16
_NW = _NC * _NS
_C = 128
_CHUNKS = 79
_EPW = _C * _CHUNKS
_EPAD = _EPW * _NC * _NS
_PAD = _EPAD - _E

_mesh = plsc.VectorSubcoreMesh(
    core_axis_name="c", subcore_axis_name="s", num_cores=_NC, num_subcores=_NS
)


_NACC = 10112
_RPT = _NACC // _NS


def _fill(ref, rows, cols, val):
    def body(i, c):
        for k in range(cols // 16):
            ref[i, pl.ds(16 * k, 16)] = jnp.full((16,), val, jnp.float32)
        return c
    lax.fori_loop(0, rows, body, 0)


def _probe_body(table, srcp, dstp, out, idx_v, dst_v, rows_v, acc_sh, sem):
    cid = lax.axis_index("c")
    sid = lax.axis_index("s")
    wid = cid * _NS + sid
    wbase = wid * _EPW

    z0 = pl.multiple_of(sid * _RPT, 8)
    _fill(rows_v, 128, _D, 0.0)
    for k in range(5):
        sz = 128 if k < 4 else _RPT - 512
        off = pl.multiple_of(z0 + k * 128, 8)
        pltpu.sync_copy(rows_v.at[pl.ds(0, sz)], acc_sh.at[pl.ds(off, sz)])
    plsc.subcore_barrier()

    def step(j, c):
        base = pl.multiple_of(wbase + j * _C, 8)
        pltpu.sync_copy(srcp.at[pl.ds(base, _C)], idx_v)
        pltpu.sync_copy(dstp.at[pl.ds(base, _C)], dst_v)
        pltpu.async_copy(table.at[idx_v], rows_v, sem).wait()
        pltpu.sync_copy(rows_v, acc_sh.at[dst_v], add=True)
        return c

    lax.fori_loop(0, _CHUNKS, step, 0)
    plsc.subcore_barrier()
    for k in range(5):
        sz = 128 if k < 4 else _RPT - 512
        off = pl.multiple_of(z0 + k * 128, 8)
        pltpu.sync_copy(acc_sh.at[pl.ds(off, sz)], rows_v.at[pl.ds(0, sz)])
        pltpu.sync_copy(rows_v.at[pl.ds(0, sz)], out.at[cid, pl.ds(off, sz)])


_probe = pl.kernel(
    _probe_body,
    out_type=jax.ShapeDtypeStruct((_NC, _NACC, _D), jnp.float32),
    mesh=_mesh,
    scratch_types=[
        pltpu.VMEM((_C,), jnp.int32),
        pltpu.VMEM((_C,), jnp.int32),
        pltpu.VMEM((_C, _D), jnp.float32),
        pltpu.VMEM_SHARED((_NACC, _D), jnp.float32),
        pltpu.SemaphoreType.DMA,
    ],
)


_R = 2000
_G = _N // _R

_DN = (((1,), (1,)), ((), ()))


def _dense_ln_body(parts, degp, h, Wl, bl, Wr, g, b, out):
    p = parts[0] + parts[1]
    deg = degp[0, :, 0:1] + degp[1, :, 0:1]
    agg = p / jnp.maximum(deg, 1.0)
    t = (lax.dot_general(agg, Wl[...], _DN, preferred_element_type=jnp.float32)
         + bl[...]
         + lax.dot_general(h[...], Wr[...], _DN, preferred_element_type=jnp.float32))
    mu = jnp.mean(t, axis=-1, keepdims=True)
    var = jnp.mean((t - mu) ** 2, axis=-1, keepdims=True)
    t = (t - mu) / jnp.sqrt(var + 1e-5) * g[...] + b[...]
    out[...] = jnp.maximum(t, 0.0) + h[...]


def _dense_fin_body(parts, degp, h, Wl, bl, Wr, out):
    p = parts[0] + parts[1]
    deg = degp[0, :, 0:1] + degp[1, :, 0:1]
    agg = p / jnp.maximum(deg, 1.0)
    out[...] = (lax.dot_general(agg, Wl[...], _DN, preferred_element_type=jnp.float32)
                + bl[...]
                + lax.dot_general(h[...], Wr[...], _DN, preferred_element_type=jnp.float32))


_spec_parts = pl.BlockSpec((_NC, _R, _D), lambda i: (0, i, 0))
_spec_degp = pl.BlockSpec((_NC, _R, 16), lambda i: (0, i, 0))
_spec_rows = pl.BlockSpec((_R, _D), lambda i: (i, 0))
_spec_w = pl.BlockSpec((_D, _D), lambda i: (0, 0))
_spec_v = pl.BlockSpec((1, _D), lambda i: (0, 0))

_dense_ln = pl.pallas_call(
    _dense_ln_body,
    grid=(_G,),
    in_specs=[_spec_parts, _spec_degp, _spec_rows, _spec_w, _spec_v,
              _spec_w, _spec_v, _spec_v],
    out_specs=_spec_rows,
    out_shape=jax.ShapeDtypeStruct((_N, _D), jnp.float32),
)

_dense_fin = pl.pallas_call(
    _dense_fin_body,
    grid=(_G,),
    in_specs=[_spec_parts, _spec_degp, _spec_rows, _spec_w, _spec_v,
              _spec_w],
    out_specs=_spec_rows,
    out_shape=jax.ShapeDtypeStruct((_N, _D), jnp.float32),
)


def kernel(x, edge_index, Wl0, bl0, Wr0, Wl1, bl1, Wr1, Wl2, bl2, Wr2,
           g0, b0, g1, b1):
    src = edge_index[0]
    dst = edge_index[1]
    ar = jnp.arange(_PAD, dtype=jnp.int32)
    srcp = jnp.concatenate([src, (ar * 37) % _N])
    dstp = jnp.concatenate([dst, _N + (ar % (_NACC - _N))])

    deg = jax.ops.segment_sum(jnp.ones((_E,), jnp.float32), dst, num_segments=_N)
    degp = jnp.zeros((_NC, _NACC, 16), jnp.float32)
    degp = degp.at[0, :_N, :].set(deg[:, None])

    bl0r, bl1r, bl2r = (v.reshape(1, _D) for v in (bl0, bl1, bl2))
    g0r, b0r, g1r, b1r = (v.reshape(1, _D) for v in (g0, b0, g1, b1))

    parts0 = _probe(x, srcp, dstp)
    h1 = _dense_ln(parts0, degp, x, Wl0, bl0r, Wr0, g0r, b0r)
    parts1 = _probe(h1, srcp, dstp)
    h2 = _dense_ln(parts1, degp, h1, Wl1, bl1r, Wr1, g1r, b1r)
    parts2 = _probe(h2, srcp, dstp)
    return _dense_fin(parts2, degp, h2, Wl2, bl2r, Wr2)


# speedup vs baseline: 12.4739x; 2.7233x over previous
import jax
import jax.numpy as jnp
from jax import lax
from jax.experimental import pallas as pl
from jax.experimental.pallas import tpu as pltpu
from jax.experimental.pallas import tpu_sc as plsc

_N = 10000
_E = 320000
_D = 128
_NC = 2
_NS = 16
_NW = _NC * _NS
_C = 128
_CHUNKS = 79
_EPW = _C * _CHUNKS
_EPAD = _EPW * _NW
_PAD = _EPAD - _E
_NACC = 10240
_RPT = _NACC // _NS

_mesh = plsc.VectorSubcoreMesh(
    core_axis_name="c", subcore_axis_name="s", num_cores=_NC, num_subcores=_NS
)


def _fill2d(ref, rows, cols, val):
    def body(i, c):
        for k in range(cols // 16):
            ref[i, pl.ds(16 * k, 16)] = jnp.full((16,), val, jnp.float32)
        return c
    lax.fori_loop(0, rows, body, 0)


def _fill1d(ref, n, val):
    def body(i, c):
        ref[pl.ds(i * 16, 16)] = jnp.full((16,), val, jnp.float32)
        return c
    lax.fori_loop(0, n // 16, body, 0)


def _zero_acc(acc_sh, buf, z0):
    for k in range(5):
        off = pl.multiple_of(z0 + k * 128, 8)
        pltpu.sync_copy(buf, acc_sh.at[pl.ds(off, 128)])


def _copy_out(acc_sh, buf, out, cid, z0):
    for k in range(5):
        off = pl.multiple_of(z0 + k * 128, 8)
        pltpu.sync_copy(acc_sh.at[pl.ds(off, 128)], buf)
        pltpu.sync_copy(buf, out.at[cid, pl.ds(off, 128)])


def _gather_scatter_loop(table, src3, dst3, acc_sh, wid,
                         src_a, src_b, dst_a, rows_a, rows_b, sem_a, sem_b,
                         deg=None):
    pltpu.sync_copy(dst3.at[wid], dst_a)
    if deg is not None:
        deg_sh, ones_v = deg

    pltpu.sync_copy(src3.at[wid, 0], src_a)
    pltpu.async_copy(table.at[src_a], rows_a, sem_a)

    def pair(jj, c):
        j0 = 2 * jj
        pltpu.sync_copy(src3.at[wid, j0 + 1], src_b)
        pltpu.async_copy(table.at[src_b], rows_b, sem_b)
        pltpu.make_async_copy(table.at[pl.ds(0, _C)], rows_a, sem_a).wait()
        pltpu.sync_copy(rows_a, acc_sh.at[dst_a.at[j0]], add=True)
        if deg is not None:
            pltpu.sync_copy(ones_v, deg_sh.at[dst_a.at[j0]], add=True)
        pltpu.sync_copy(src3.at[wid, j0 + 2], src_a)
        pltpu.async_copy(table.at[src_a], rows_a, sem_a)
        pltpu.make_async_copy(table.at[pl.ds(0, _C)], rows_b, sem_b).wait()
        pltpu.sync_copy(rows_b, acc_sh.at[dst_a.at[j0 + 1]], add=True)
        if deg is not None:
            pltpu.sync_copy(ones_v, deg_sh.at[dst_a.at[j0 + 1]], add=True)
        return c

    lax.fori_loop(0, (_CHUNKS - 1) // 2, pair, 0)
    pltpu.make_async_copy(table.at[pl.ds(0, _C)], rows_a, sem_a).wait()
    pltpu.sync_copy(rows_a, acc_sh.at[dst_a.at[_CHUNKS - 1]], add=True)
    if deg is not None:
        pltpu.sync_copy(ones_v, deg_sh.at[dst_a.at[_CHUNKS - 1]], add=True)


def _agg_deg_body(table, src3, dst3, parts, degp,
                  src_a, src_b, dst_a, rows_a, rows_b, ones_v, vec_v,
                  acc_sh, deg_sh, sem_a, sem_b):
    cid = lax.axis_index("c")
    sid = lax.axis_index("s")
    wid = cid * _NS + sid
    z0 = pl.multiple_of(sid * _RPT, 8)
    _fill2d(rows_a, 128, _D, 0.0)
    _zero_acc(acc_sh, rows_a, z0)
    _fill1d(vec_v, _RPT, 0.0)
    pltpu.sync_copy(vec_v, deg_sh.at[pl.ds(z0, _RPT)])
    _fill1d(ones_v, _C, 1.0)
    plsc.subcore_barrier()

    _gather_scatter_loop(table, src3, dst3, acc_sh, wid,
                         src_a, src_b, dst_a, rows_a, rows_b, sem_a, sem_b,
                         deg=(deg_sh, ones_v))
    plsc.subcore_barrier()

    _copy_out(acc_sh, rows_b, parts, cid, z0)
    pltpu.sync_copy(deg_sh.at[pl.ds(z0, _RPT)], vec_v)
    pltpu.sync_copy(vec_v, degp.at[cid, pl.ds(z0, _RPT)])


def _agg_body(table, src3, dst3, parts,
              src_a, src_b, dst_a, rows_a, rows_b, acc_sh, sem_a, sem_b):
    cid = lax.axis_index("c")
    sid = lax.axis_index("s")
    wid = cid * _NS + sid
    z0 = pl.multiple_of(sid * _RPT, 8)
    _fill2d(rows_a, 128, _D, 0.0)
    _zero_acc(acc_sh, rows_a, z0)
    plsc.subcore_barrier()

    _gather_scatter_loop(table, src3, dst3, acc_sh, wid,
                         src_a, src_b, dst_a, rows_a, rows_b, sem_a, sem_b)
    plsc.subcore_barrier()

    _copy_out(acc_sh, rows_b, parts, cid, z0)


_agg_deg = pl.kernel(
    _agg_deg_body,
    out_type=(
        jax.ShapeDtypeStruct((_NC, _NACC, _D), jnp.float32),
        jax.ShapeDtypeStruct((_NC, _NACC), jnp.float32),
    ),
    mesh=_mesh,
    scratch_types=[
        pltpu.VMEM((_C,), jnp.int32),
        pltpu.VMEM((_C,), jnp.int32),
        pltpu.VMEM((_CHUNKS, _C), jnp.int32),
        pltpu.VMEM((_C, _D), jnp.float32),
        pltpu.VMEM((_C, _D), jnp.float32),
        pltpu.VMEM((_C,), jnp.float32),
        pltpu.VMEM((_RPT,), jnp.float32),
        pltpu.VMEM_SHARED((_NACC, _D), jnp.float32),
        pltpu.VMEM_SHARED((_NACC,), jnp.float32),
        pltpu.SemaphoreType.DMA,
        pltpu.SemaphoreType.DMA,
    ],
)

_agg = pl.kernel(
    _agg_body,
    out_type=jax.ShapeDtypeStruct((_NC, _NACC, _D), jnp.float32),
    mesh=_mesh,
    scratch_types=[
        pltpu.VMEM((_C,), jnp.int32),
        pltpu.VMEM((_C,), jnp.int32),
        pltpu.VMEM((_CHUNKS, _C), jnp.int32),
        pltpu.VMEM((_C, _D), jnp.float32),
        pltpu.VMEM((_C, _D), jnp.float32),
        pltpu.VMEM_SHARED((_NACC, _D), jnp.float32),
        pltpu.SemaphoreType.DMA,
        pltpu.SemaphoreType.DMA,
    ],
)


_R = 2000
_G = _N // _R

_DN = (((1,), (1,)), ((), ()))


def _dense_ln_body(parts, deg, h, Wl, bl, Wr, g, b, out):
    p = parts[0] + parts[1]
    agg = p / jnp.maximum(deg[...], 1.0)
    t = (lax.dot_general(agg, Wl[...], _DN, preferred_element_type=jnp.float32)
         + bl[...]
         + lax.dot_general(h[...], Wr[...], _DN, preferred_element_type=jnp.float32))
    mu = jnp.mean(t, axis=-1, keepdims=True)
    var = jnp.mean((t - mu) ** 2, axis=-1, keepdims=True)
    t = (t - mu) / jnp.sqrt(var + 1e-5) * g[...] + b[...]
    out[...] = jnp.maximum(t, 0.0) + h[...]


def _dense_fin_body(parts, deg, h, Wl, bl, Wr, out):
    p = parts[0] + parts[1]
    agg = p / jnp.maximum(deg[...], 1.0)
    out[...] = (lax.dot_general(agg, Wl[...], _DN, preferred_element_type=jnp.float32)
                + bl[...]
                + lax.dot_general(h[...], Wr[...], _DN, preferred_element_type=jnp.float32))


_spec_parts = pl.BlockSpec((_NC, _R, _D), lambda i: (0, i, 0))
_spec_deg = pl.BlockSpec((_R, 1), lambda i: (i, 0))
_spec_rows = pl.BlockSpec((_R, _D), lambda i: (i, 0))
_spec_w = pl.BlockSpec((_D, _D), lambda i: (0, 0))
_spec_v = pl.BlockSpec((1, _D), lambda i: (0, 0))

_dense_ln = pl.pallas_call(
    _dense_ln_body,
    grid=(_G,),
    in_specs=[_spec_parts, _spec_deg, _spec_rows, _spec_w, _spec_v,
              _spec_w, _spec_v, _spec_v],
    out_specs=_spec_rows,
    out_shape=jax.ShapeDtypeStruct((_N, _D), jnp.float32),
)

_dense_fin = pl.pallas_call(
    _dense_fin_body,
    grid=(_G,),
    in_specs=[_spec_parts, _spec_deg, _spec_rows, _spec_w, _spec_v,
              _spec_w],
    out_specs=_spec_rows,
    out_shape=jax.ShapeDtypeStruct((_N, _D), jnp.float32),
)


def kernel(x, edge_index, Wl0, bl0, Wr0, Wl1, bl1, Wr1, Wl2, bl2, Wr2,
           g0, b0, g1, b1):
    src = edge_index[0]
    dst = edge_index[1]
    ar = jnp.arange(_PAD, dtype=jnp.int32)
    src3 = jnp.concatenate([src, (ar * 37) % _N]).reshape(_NW, _CHUNKS, _C)
    dst3 = jnp.concatenate([dst, _N + (ar % (_NACC - _N))]).reshape(_NW, _CHUNKS, _C)

    bl0r, bl1r, bl2r = (v.reshape(1, _D) for v in (bl0, bl1, bl2))
    g0r, b0r, g1r, b1r = (v.reshape(1, _D) for v in (g0, b0, g1, b1))

    parts0, degp = _agg_deg(x, src3, dst3)
    deg = (degp[0, :_N] + degp[1, :_N]).reshape(_N, 1)
    h1 = _dense_ln(parts0, deg, x, Wl0, bl0r, Wr0, g0r, b0r)
    parts1 = _agg(h1, src3, dst3)
    h2 = _dense_ln(parts1, deg, h1, Wl1, bl1r, Wr1, g1r, b1r)
    parts2 = _agg(h2, src3, dst3)
    return _dense_fin(parts2, deg, h2, Wl2, bl2r, Wr2)
